# baseline (speedup 1.0000x reference)
import jax
import jax.numpy as jnp
from jax import lax
from jax.experimental import pallas as pl
from jax.experimental.pallas import tpu as pltpu

SEQ = 2048
DIM = 4096
NCHUNK = 8
CROWS = SEQ // NCHUNK


def _body(mv_ref, ms_ref, a_hbm, x_hbm, o_hbm, sem, sem2):
    copies = [
        pltpu.make_async_copy(
            a_hbm.at[pl.ds(k * CROWS, CROWS), :],
            o_hbm.at[pl.ds(k * CROWS, CROWS), :], sem)
        for k in range(NCHUNK)
    ]
    for cp in copies:
        cp.start()
    any_false = jnp.min(mv_ref[...]) == 0
    for cp in copies:
        cp.wait()

    @pl.when(any_false)
    def _():
        def fix(r, carry):
            @pl.when(ms_ref[r] == 0)
            def _():
                rcp = pltpu.make_async_copy(
                    x_hbm.at[pl.ds(r, 1), :], o_hbm.at[pl.ds(r, 1), :], sem2)
                rcp.start()
                rcp.wait()
            return carry

        lax.fori_loop(0, SEQ, fix, 0)


def kernel(x, attack, attack_mask):
    x2 = x.reshape(SEQ, DIM)
    a2 = attack.reshape(SEQ, DIM)
    mi = attack_mask.reshape(SEQ).astype(jnp.int32)
    mv = mi.reshape(16, 128)
    out = pl.pallas_call(
        _body,
        in_specs=[
            pl.BlockSpec(memory_space=pltpu.MemorySpace.VMEM),
            pl.BlockSpec(memory_space=pltpu.MemorySpace.SMEM),
            pl.BlockSpec(memory_space=pltpu.MemorySpace.HBM),
            pl.BlockSpec(memory_space=pltpu.MemorySpace.HBM),
        ],
        out_specs=pl.BlockSpec(memory_space=pltpu.MemorySpace.HBM),
        out_shape=jax.ShapeDtypeStruct((SEQ, DIM), x.dtype),
        scratch_shapes=[
            pltpu.SemaphoreType.DMA,
            pltpu.SemaphoreType.DMA,
        ],
    )(mv, mi, a2, x2)
    return out.reshape(1, SEQ, DIM)


# speedup vs baseline: 43.1138x; 43.1138x over previous
import jax
import jax.numpy as jnp
from jax.experimental import pallas as pl
from jax.experimental.pallas import tpu as pltpu

SEQ = 2048
DIM = 4096
BLK = 512
NBLK = SEQ // BLK


def _body(m_ref, a_ref, x_hbm, o_ref, x_vmem, sem):
    i = pl.program_id(0)
    need_x = jnp.any(m_ref[...] == 0)

    @pl.when(need_x)
    def _():
        cp = pltpu.make_async_copy(
            x_hbm.at[pl.ds(i * BLK, BLK), :], x_vmem, sem)
        cp.start()
        cp.wait()
        o_ref[...] = jnp.where(m_ref[...] != 0, a_ref[...], x_vmem[...])

    @pl.when(jnp.logical_not(need_x))
    def _():
        o_ref[...] = a_ref[...]


def kernel(x, attack, attack_mask):
    x2 = x.reshape(SEQ, DIM)
    a2 = attack.reshape(SEQ, DIM)
    m2 = attack_mask.reshape(SEQ, 1).astype(jnp.int32)
    out = pl.pallas_call(
        _body,
        grid=(NBLK,),
        in_specs=[
            pl.BlockSpec((BLK, 1), lambda i: (i, 0)),
            pl.BlockSpec((BLK, DIM), lambda i: (i, 0)),
            pl.BlockSpec(memory_space=pltpu.MemorySpace.HBM),
        ],
        out_specs=pl.BlockSpec((BLK, DIM), lambda i: (i, 0)),
        out_shape=jax.ShapeDtypeStruct((SEQ, DIM), x.dtype),
        scratch_shapes=[
            pltpu.VMEM((BLK, DIM), jnp.float32),
            pltpu.SemaphoreType.DMA,
        ],
    )(m2, a2, x2)
    return out.reshape(1, SEQ, DIM)
